# baseline (speedup 1.0000x reference)
import functools

import jax
import jax.numpy as jnp
from jax import lax
from jax.experimental import pallas as pl
from jax.experimental.pallas import tpu as pltpu
from jax.experimental.pallas import tpu_sc as plsc

W, L, H = 256, 256, 16
V = W * L * H
MINPTS = 10
NB = 2
C = 8
N = 64 * 2048
NSUB = 16
VPT = V // NSUB
ST = 4096
NST = VPT // ST
CH_A = 1024
CH_B = 4096
PPT = N // NSUB
QTRIG = 9216
QALLOC = QTRIG + CH_B + 288
CCAP = 1024
HASHN = 2048
GROW = 128


def _i1():
    return jnp.full((16,), 1, jnp.int32)


def _i0():
    return jnp.zeros((16,), jnp.int32)


def _mask_i32(m):
    return jnp.where(m, _i1(), _i0())


def _body(pc_ref, pa_ref, prm_ref,
          vox_ref, occ_ref, cnt_hbm, idx_hbm, pat_ref,
          pcx, pcy, pcz, iob, pab, sbuf, qrel, qpid, srtR, srtP,
          slab, cnt, occb, tmp, hist, ptrv, abuf, pbuf, vsbuf,
          smem, sem, gsem):
    b = lax.axis_index("c")
    t = lax.axis_index("s")
    lo = t * VPT
    lane = lax.broadcasted_iota(jnp.int32, (16,), 0)

    pltpu.sync_copy(prm_ref.at[pl.ds(b * 48, 48)], pbuf)
    pltpu.sync_copy(prm_ref.at[pl.ds(NB * 48, 16)], vsbuf)
    oxv = pbuf[pl.ds(0, 16)]
    oyv = pbuf[pl.ds(16, 16)]
    ozv = pbuf[pl.ds(32, 16)]
    vsv = vsbuf[...]

    smem[0] = 1
    smem[1] = -1
    smem[2] = 0
    def _zc(j, _):
        srtP[pl.ds(j * 16, 16)] = _i0()
        return 0
    lax.fori_loop(0, QALLOC // 16, _zc, 0)
    def _zd(j, _):
        smem[3 + j] = 0
        return 0
    lax.fori_loop(0, NST, _zd, 0)

    pbase = t * PPT
    for sc_i in range(PPT // CH_A):
        base = pbase + sc_i * CH_A
        pltpu.sync_copy(pc_ref.at[pl.ds((b * 3 + 0) * N + base, CH_A)], pcx)
        pltpu.sync_copy(pc_ref.at[pl.ds((b * 3 + 1) * N + base, CH_A)], pcy)
        pltpu.sync_copy(pc_ref.at[pl.ds((b * 3 + 2) * N + base, CH_A)], pcz)
        for c in range(C):
            pltpu.make_async_copy(
                pa_ref.at[pl.ds((b * C + c) * N + base, CH_A)],
                pab.at[pl.ds(c * CH_A, CH_A)], gsem).start()

        def _abody(j, _):
            s = pl.ds(j * 16, 16)
            ix = ((pcx[s] - oxv) / vsv).astype(jnp.int32)
            iy = ((pcy[s] - oyv) / vsv).astype(jnp.int32)
            iz = ((pcz[s] - ozv) / vsv).astype(jnp.int32)
            m = ((ix >= 0) & (ix < W) & (iy >= 0) & (iy < L)
                 & (iz >= 0) & (iz < H))
            flat = ix * (L * H) + iy * H + iz
            iob[s] = jnp.where(m, flat, jnp.full((16,), V, jnp.int32))
            return 0
        lax.fori_loop(0, CH_A // 16, _abody, 0)
        pltpu.sync_copy(iob, idx_hbm.at[pl.ds(b * N + base, CH_A)])

        for c in range(C):
            pltpu.make_async_copy(
                pa_ref.at[pl.ds((b * C + c) * N + base, CH_A)],
                pab.at[pl.ds(c * CH_A, CH_A)], gsem).wait()

        def _tbody(j, _):
            rows = lane + j * 16
            for c in range(C):
                v = pab[pl.ds(c * CH_A + j * 16, 16)]
                plsc.store_scatter(abuf, [rows, jnp.full((16,), c, jnp.int32)],
                                   v)
            return 0
        lax.fori_loop(0, CH_A // 16, _tbody, 0)
        pltpu.sync_copy(abuf, pat_ref.at[pl.ds(b * N + base, CH_A)])
    plsc.subcore_barrier()

    def _out_copies(stprev, final):
        x = t * NST + stprev
        cps = []
        for c in range(C):
            cps.append((slab.at[c], vox_ref.at[b, c, x]))
        cps.append((cnt, cnt_hbm.at[pl.ds(b * V + x * ST, ST)]))
        if final:
            cps.append((occb, occ_ref.at[b, 0, x]))
        return cps

    def _drain_prev():
        stprev = smem[1]

        @pl.when(stprev >= 0)
        def _():
            was_final = smem[2]

            @pl.when(was_final == 0)
            def _():
                for src, dst in _out_copies(stprev, False):
                    pltpu.make_async_copy(src, dst, sem).wait()

            @pl.when(was_final != 0)
            def _():
                for src, dst in _out_copies(stprev, True):
                    pltpu.make_async_copy(src, dst, sem).wait()
            smem[1] = -1

    def _ensure_slab(st):
        @pl.when(smem[0] == 0)
        def _():
            _drain_prev()
            x = t * NST + st

            @pl.when(smem[3 + st] != 0)
            def _():
                for c in range(C):
                    pltpu.sync_copy(vox_ref.at[b, c, x], slab.at[c])
                pltpu.sync_copy(cnt_hbm.at[pl.ds(b * V + x * ST, ST)], cnt)

            @pl.when(smem[3 + st] == 0)
            def _():
                zf = jnp.zeros((16,), jnp.float32)

                def _zs(j, _):
                    zj = lax.shift_right_logical(j, 4)
                    yg = (j & 15) * 16
                    for c in range(C):
                        slab[c, zj, pl.ds(yg, 16)] = zf
                    return 0
                lax.fori_loop(0, (H * L) // 16, _zs, 0)

                def _zc2(j, _):
                    for u in range(8):
                        cnt[pl.ds(j * 128 + u * 16, 16)] = _i0()
                    return 0
                lax.fori_loop(0, ST // 128, _zc2, 0)
            smem[0] = 1

    def _process_chunk(cbase, cn, st):
        _ensure_slab(st)
        ng = (cn + (GROW - 1)) // GROW

        def _gs(g, _):
            pltpu.make_async_copy(
                pat_ref.at[srtP.at[pl.ds(cbase + g * GROW, GROW)]],
                abuf.at[pl.ds(g * GROW, GROW)], gsem).start()
            return 0
        lax.fori_loop(0, ng, _gs, 0)

        def _gw(g, _):
            pltpu.make_async_copy(
                pat_ref.at[srtP.at[pl.ds(cbase + g * GROW, GROW)]],
                abuf.at[pl.ds(g * GROW, GROW)], gsem).wait()
            return 0
        lax.fori_loop(0, ng, _gw, 0)

        def _acc(j, _):
            srel = srtR[pl.ds(cbase + j * 16, 16)] - st * ST
            valid = (lane + j * 16) < cn
            row = lane + j * 16

            def _wcond(remi):
                return jnp.sum(remi) > 0

            def _wbody(remi):
                rem = remi > 0
                hsh = srel & (HASHN - 1)
                plsc.store_scatter(tmp, [hsh], lane, mask=rem)
                win = plsc.load_gather(tmp, [hsh], mask=rem)
                my = rem & (win == lane)
                cv = plsc.load_gather(cnt, [srel], mask=my)
                plsc.store_scatter(cnt, [srel], cv + 1, mask=my)
                yy = lax.shift_right_logical(srel, 4)
                zz = srel & 15
                for c in range(C):
                    cc = jnp.full((16,), c, jnp.int32)
                    av = plsc.load_gather(abuf, [row, cc], mask=my)
                    sv = plsc.load_gather(slab, [cc, zz, yy], mask=my)
                    plsc.store_scatter(slab, [cc, zz, yy],
                                       jnp.maximum(sv, av), mask=my)
                return jnp.where(my, _i0(), remi)

            lax.while_loop(_wcond, _wbody, _mask_i32(valid))
            return 0
        lax.fori_loop(0, (cn + 15) // 16, _acc, 0)
        return jnp.int32(0)

    def _flush(qn, final):
        hist[...] = _i0()
        nv = (qn + 15) // 16

        big = jnp.full((16,), 1 << 20, jnp.int32)

        def _runinfo(rel, valid):
            stv = lax.shift_right_logical(rel, 12)
            key = jnp.where(valid, stv * 32 + lane, big)
            ks, vs = plsc.sort_key_val(key, lane)
            sstv = lax.shift_right_logical(ks, 5)
            tmp[pl.ds(0, 16)] = sstv
            prev = plsc.load_gather(tmp, [jnp.maximum(lane - 1, _i0())])
            nxt = plsc.load_gather(tmp, [jnp.minimum(lane + 1,
                                                     jnp.full((16,), 15,
                                                              jnp.int32))])
            svalid = ks < big
            bnd = (lane == 0) | (sstv != prev)
            last = svalid & ((lane == 15) | (sstv != nxt))
            run_start = plsc.cummax(jnp.where(bnd, lane, _i0()))
            runlen = lane - run_start + 1
            return stv, vs, sstv, svalid, last, run_start, runlen

        def _hb(j, _):
            rel = qrel[pl.ds(j * 16, 16)]
            valid = (lane + j * 16) < qn
            _, _, sstv, _, last, _, runlen = _runinfo(rel, valid)
            hv = plsc.load_gather(hist, [sstv], mask=last)
            plsc.store_scatter(hist, [sstv], hv + runlen, mask=last)
            return 0
        lax.fori_loop(0, nv, _hb, 0)

        h = hist[...]
        hp = (h + 15) & jnp.full((16,), ~15, jnp.int32)
        off = plsc.cumsum(hp) - hp
        ptrv[...] = off

        def _sg(st, _):
            cs = jnp.sum(jnp.where(lane == st, h, _i0()))
            os_ = jnp.sum(jnp.where(lane == st, off, _i0()))
            smem[3 + NST + st] = cs
            smem[3 + 2 * NST + st] = os_
            return 0
        lax.fori_loop(0, NST, _sg, 0)

        def _pb(j, _):
            rel = qrel[pl.ds(j * 16, 16)]
            pid = qpid[pl.ds(j * 16, 16)]
            valid = (lane + j * 16) < qn
            _, vs, sstv, svalid, last, run_start, runlen = _runinfo(rel, valid)
            base = plsc.load_gather(ptrv, [sstv], mask=svalid)
            pos = base + (lane - run_start)
            plsc.store_scatter(ptrv, [sstv], base + runlen, mask=last)
            plsc.store_scatter(tmp, [vs + 16], pos, mask=svalid)
            posv = plsc.load_gather(tmp, [lane + 16], mask=valid)
            plsc.store_scatter(srtR, [posv], rel, mask=valid)
            plsc.store_scatter(srtP, [posv], pid, mask=valid)
            return 0
        lax.fori_loop(0, nv, _pb, 0)

        def _stbody(st, _):
            smem[0] = 0
            cnt_st = smem[3 + NST + st]
            off_st = pl.multiple_of(smem[3 + 2 * NST + st], 16)
            nchunks = (cnt_st + CCAP - 1) // CCAP

            def _ck(g, _):
                cbase = off_st + g * CCAP
                cn = jnp.minimum(cnt_st - g * CCAP, CCAP)
                _process_chunk(cbase, cn, st)
                return 0
            lax.fori_loop(0, nchunks, _ck, 0)
            _ensure_slab(st)
            if final:
                def _ob(j, _):
                    zj = lax.shift_right_logical(j, 4)
                    yg = (j & 15) * 16
                    cv = plsc.load_gather(cnt, [(lane + yg) * H + zj])
                    occb[zj, pl.ds(yg, 16)] = jnp.where(
                        cv >= MINPTS, jnp.full((16,), 1.0, jnp.float32),
                        jnp.zeros((16,), jnp.float32))
                    return 0
                lax.fori_loop(0, (H * L) // 16, _ob, 0)
            for src, dst in _out_copies(st, final):
                pltpu.make_async_copy(src, dst, sem).start()
            smem[1] = st
            smem[2] = jnp.int32(1 if final else 0)
            smem[3 + st] = 1
            return 0
        lax.fori_loop(0, NST, _stbody, 0)
        return jnp.int32(0)

    def _chunk(k, qnv):
        pltpu.sync_copy(idx_hbm.at[pl.ds(b * N + k * CH_B, CH_B)], sbuf)

        def _fbody(j, qnv):
            for u in range(4):
                s = pl.ds((j * 4 + u) * 16, 16)
                rel = sbuf[s] - lo
                m = (rel >= 0) & (rel < VPT)
                pidv = lane + (b * N + k * CH_B + (j * 4 + u) * 16)
                mi = _mask_i32(m)
                inc = plsc.cumsum(mi)
                pos = qnv + inc - mi
                plsc.store_scatter(qrel, [pos], rel, mask=m)
                plsc.store_scatter(qpid, [pos], pidv, mask=m)
                qnv = qnv + plsc.all_reduce_population_count(m)
            return qnv
        qnv = lax.fori_loop(0, CH_B // 64, _fbody, qnv)
        qs = jnp.max(qnv)

        def _doflush(q):
            _flush(q, False)
            return _i0()
        qnv = lax.cond(qs > QTRIG, _doflush, lambda q: qnv, qs)
        return qnv

    qnv = lax.fori_loop(0, N // CH_B, _chunk, _i0())
    _flush(jnp.max(qnv), True)
    _drain_prev()


@functools.partial(jax.jit, static_argnums=())
def _run(pc1, pa1, prm):
    mesh = plsc.VectorSubcoreMesh(core_axis_name="c", subcore_axis_name="s")
    f = pl.kernel(
        _body,
        mesh=mesh,
        compiler_params=pltpu.CompilerParams(
            needs_layout_passes=False, use_tc_tiling_on_sc=False),
        out_type=[
            jax.ShapeDtypeStruct((NB, C, W, H, L), jnp.float32),
            jax.ShapeDtypeStruct((NB, 1, W, H, L), jnp.float32),
            jax.ShapeDtypeStruct((NB * V,), jnp.int32),
            jax.ShapeDtypeStruct((NB * N,), jnp.int32),
            jax.ShapeDtypeStruct((NB * N, C), jnp.float32),
        ],
        scratch_types=[
            pltpu.VMEM((CH_A,), jnp.float32),
            pltpu.VMEM((CH_A,), jnp.float32),
            pltpu.VMEM((CH_A,), jnp.float32),
            pltpu.VMEM((CH_A,), jnp.int32),
            pltpu.VMEM((C * CH_A,), jnp.float32),
            pltpu.VMEM((CH_B,), jnp.int32),
            pltpu.VMEM((QALLOC,), jnp.int32),
            pltpu.VMEM((QALLOC,), jnp.int32),
            pltpu.VMEM((QALLOC,), jnp.int32),
            pltpu.VMEM((QALLOC,), jnp.int32),
            pltpu.VMEM((C, H, L), jnp.float32),
            pltpu.VMEM((ST,), jnp.int32),
            pltpu.VMEM((H, L), jnp.float32),
            pltpu.VMEM((HASHN,), jnp.int32),
            pltpu.VMEM((16,), jnp.int32),
            pltpu.VMEM((16,), jnp.int32),
            pltpu.VMEM((CCAP, C), jnp.float32),
            pltpu.VMEM((48,), jnp.float32),
            pltpu.VMEM((16,), jnp.float32),
            pltpu.SMEM((3 + 3 * NST,), jnp.int32),
            pltpu.SemaphoreType.DMA,
            pltpu.SemaphoreType.DMA,
        ],
    )
    return f(pc1, pa1, prm)


def kernel(point_coordinates, point_attributes, origin, voxel_size):
    b, _, rh, rw = point_coordinates.shape
    c = point_attributes.shape[1]
    assert (b, c, rh * rw) == (NB, C, N)
    pc1 = point_coordinates.reshape(b * 3 * N)
    pa1 = point_attributes.reshape(b * c * N)
    prm = jnp.concatenate([
        jnp.broadcast_to(origin.reshape(b * 3, 1), (b * 3, 16)),
        jnp.broadcast_to(voxel_size.reshape(1, 1), (1, 16)),
    ], axis=0).reshape(-1)
    vox, occ, _cnt, _idx, _pat = _run(pc1, pa1, prm)
    return (jnp.transpose(vox, (0, 1, 2, 4, 3)),
            jnp.transpose(occ, (0, 1, 2, 4, 3)))


# speedup vs baseline: 1.0495x; 1.0495x over previous
import functools

import jax
import jax.numpy as jnp
from jax import lax
from jax.experimental import pallas as pl
from jax.experimental.pallas import tpu as pltpu
from jax.experimental.pallas import tpu_sc as plsc

W, L, H = 256, 256, 16
V = W * L * H
MINPTS = 10
NB = 2
C = 8
N = 64 * 2048
NSUB = 16
VPT = V // NSUB
ST = 4096
NST = VPT // ST
CH_A = 1024
CH_B = 4096
PPT = N // NSUB
QTRIG = 9216
QALLOC = QTRIG + CH_B + 288
CCAP = 1024
HASHN = 2048
GROW = 128


def _i1():
    return jnp.full((16,), 1, jnp.int32)


def _i0():
    return jnp.zeros((16,), jnp.int32)


def _mask_i32(m):
    return jnp.where(m, _i1(), _i0())


def _body(pc_ref, pa_ref, prm_ref,
          vox_ref, occ_ref, cnt_hbm, idx_hbm, pat_ref,
          pcx, pcy, pcz, iob, pab, sbuf, qrel, qpid, srtR, srtP,
          slab, cnt, occb, tmp, hist, ptrv, abuf, pbuf, vsbuf,
          smem, sem, gsem, ssem):
    b = lax.axis_index("c")
    t = lax.axis_index("s")
    lo = t * VPT
    lane = lax.broadcasted_iota(jnp.int32, (16,), 0)

    pltpu.sync_copy(prm_ref.at[pl.ds(b * 48, 48)], pbuf)
    pltpu.sync_copy(prm_ref.at[pl.ds(NB * 48, 16)], vsbuf)
    oxv = pbuf[pl.ds(0, 16)]
    oyv = pbuf[pl.ds(16, 16)]
    ozv = pbuf[pl.ds(32, 16)]
    vsv = vsbuf[...]

    smem[0] = 1
    smem[1] = -1
    smem[2] = 0
    def _zc(j, _):
        srtP[pl.ds(j * 16, 16)] = _i0()
        return 0
    lax.fori_loop(0, QALLOC // 16, _zc, 0)
    def _zd(j, _):
        smem[3 + j] = 0
        return 0
    lax.fori_loop(0, NST, _zd, 0)

    pbase = t * PPT
    for sc_i in range(PPT // CH_A):
        base = pbase + sc_i * CH_A
        pltpu.sync_copy(pc_ref.at[pl.ds((b * 3 + 0) * N + base, CH_A)], pcx)
        pltpu.sync_copy(pc_ref.at[pl.ds((b * 3 + 1) * N + base, CH_A)], pcy)
        pltpu.sync_copy(pc_ref.at[pl.ds((b * 3 + 2) * N + base, CH_A)], pcz)
        for c in range(C):
            pltpu.make_async_copy(
                pa_ref.at[pl.ds((b * C + c) * N + base, CH_A)],
                pab.at[pl.ds(c * CH_A, CH_A)], gsem).start()

        def _abody(j, _):
            s = pl.ds(j * 16, 16)
            ix = ((pcx[s] - oxv) / vsv).astype(jnp.int32)
            iy = ((pcy[s] - oyv) / vsv).astype(jnp.int32)
            iz = ((pcz[s] - ozv) / vsv).astype(jnp.int32)
            m = ((ix >= 0) & (ix < W) & (iy >= 0) & (iy < L)
                 & (iz >= 0) & (iz < H))
            flat = ix * (L * H) + iy * H + iz
            iob[s] = jnp.where(m, flat, jnp.full((16,), V, jnp.int32))
            return 0
        lax.fori_loop(0, CH_A // 16, _abody, 0)
        pltpu.sync_copy(iob, idx_hbm.at[pl.ds(b * N + base, CH_A)])

        for c in range(C):
            pltpu.make_async_copy(
                pa_ref.at[pl.ds((b * C + c) * N + base, CH_A)],
                pab.at[pl.ds(c * CH_A, CH_A)], gsem).wait()

        def _tbody(j, _):
            rows = lane + j * 16
            for c in range(C):
                v = pab[pl.ds(c * CH_A + j * 16, 16)]
                plsc.store_scatter(abuf, [rows, jnp.full((16,), c, jnp.int32)],
                                   v)
            return 0
        lax.fori_loop(0, CH_A // 16, _tbody, 0)
        pltpu.sync_copy(abuf, pat_ref.at[pl.ds(b * N + base, CH_A)])
    plsc.subcore_barrier()

    def _out_copies(stprev, final):
        x = t * NST + stprev
        cps = []
        for c in range(C):
            cps.append((slab.at[c], vox_ref.at[b, c, x]))
        cps.append((cnt, cnt_hbm.at[pl.ds(b * V + x * ST, ST)]))
        if final:
            cps.append((occb, occ_ref.at[b, 0, x]))
        return cps

    def _drain_prev():
        stprev = smem[1]

        @pl.when(stprev >= 0)
        def _():
            was_final = smem[2]

            @pl.when(was_final == 0)
            def _():
                for src, dst in _out_copies(stprev, False):
                    pltpu.make_async_copy(src, dst, sem).wait()

            @pl.when(was_final != 0)
            def _():
                for src, dst in _out_copies(stprev, True):
                    pltpu.make_async_copy(src, dst, sem).wait()
            smem[1] = -1

    def _ensure_slab(st):
        @pl.when(smem[0] == 0)
        def _():
            _drain_prev()
            x = t * NST + st

            @pl.when(smem[3 + st] != 0)
            def _():
                for c in range(C):
                    pltpu.sync_copy(vox_ref.at[b, c, x], slab.at[c])
                pltpu.sync_copy(cnt_hbm.at[pl.ds(b * V + x * ST, ST)], cnt)

            @pl.when(smem[3 + st] == 0)
            def _():
                zf = jnp.zeros((16,), jnp.float32)

                def _zs(j, _):
                    zj = lax.shift_right_logical(j, 4)
                    yg = (j & 15) * 16
                    for c in range(C):
                        slab[c, zj, pl.ds(yg, 16)] = zf
                    return 0
                lax.fori_loop(0, (H * L) // 16, _zs, 0)

                def _zc2(j, _):
                    for u in range(8):
                        cnt[pl.ds(j * 128 + u * 16, 16)] = _i0()
                    return 0
                lax.fori_loop(0, ST // 128, _zc2, 0)
            smem[0] = 1

    def _process_chunk(cbase, cn, st):
        _ensure_slab(st)
        ng = (cn + (GROW - 1)) // GROW

        def _gs(g, _):
            pltpu.make_async_copy(
                pat_ref.at[srtP.at[pl.ds(cbase + g * GROW, GROW)]],
                abuf.at[pl.ds(g * GROW, GROW)], gsem).start()
            return 0
        lax.fori_loop(0, ng, _gs, 0)

        def _gw(g, _):
            pltpu.make_async_copy(
                pat_ref.at[srtP.at[pl.ds(cbase + g * GROW, GROW)]],
                abuf.at[pl.ds(g * GROW, GROW)], gsem).wait()
            return 0
        lax.fori_loop(0, ng, _gw, 0)

        def _acc(j, _):
            srel = srtR[pl.ds(cbase + j * 16, 16)] - st * ST
            valid = (lane + j * 16) < cn
            row = lane + j * 16

            def _wcond(remi):
                return jnp.sum(remi) > 0

            def _wbody(remi):
                rem = remi > 0
                hsh = srel & (HASHN - 1)
                plsc.store_scatter(tmp, [hsh], lane, mask=rem)
                win = plsc.load_gather(tmp, [hsh], mask=rem)
                my = rem & (win == lane)
                cv = plsc.load_gather(cnt, [srel], mask=my)
                plsc.store_scatter(cnt, [srel], cv + 1, mask=my)
                yy = lax.shift_right_logical(srel, 4)
                zz = srel & 15
                for c in range(C):
                    cc = jnp.full((16,), c, jnp.int32)
                    av = plsc.load_gather(abuf, [row, cc], mask=my)
                    sv = plsc.load_gather(slab, [cc, zz, yy], mask=my)
                    plsc.store_scatter(slab, [cc, zz, yy],
                                       jnp.maximum(sv, av), mask=my)
                return jnp.where(my, _i0(), remi)

            lax.while_loop(_wcond, _wbody, _mask_i32(valid))
            return 0
        lax.fori_loop(0, (cn + 15) // 16, _acc, 0)
        return jnp.int32(0)

    def _flush(qn, final):
        hist[...] = _i0()
        nv = (qn + 15) // 16

        big = jnp.full((16,), 1 << 20, jnp.int32)

        def _runinfo(rel, valid):
            stv = lax.shift_right_logical(rel, 12)
            key = jnp.where(valid, stv * 32 + lane, big)
            ks, vs = plsc.sort_key_val(key, lane)
            sstv = lax.shift_right_logical(ks, 5)
            tmp[pl.ds(0, 16)] = sstv
            prev = plsc.load_gather(tmp, [jnp.maximum(lane - 1, _i0())])
            nxt = plsc.load_gather(tmp, [jnp.minimum(lane + 1,
                                                     jnp.full((16,), 15,
                                                              jnp.int32))])
            svalid = ks < big
            bnd = (lane == 0) | (sstv != prev)
            last = svalid & ((lane == 15) | (sstv != nxt))
            run_start = plsc.cummax(jnp.where(bnd, lane, _i0()))
            runlen = lane - run_start + 1
            return stv, vs, sstv, svalid, last, run_start, runlen

        def _hb(j, _):
            rel = qrel[pl.ds(j * 16, 16)]
            valid = (lane + j * 16) < qn
            _, _, sstv, _, last, _, runlen = _runinfo(rel, valid)
            hv = plsc.load_gather(hist, [sstv], mask=last)
            plsc.store_scatter(hist, [sstv], hv + runlen, mask=last)
            return 0
        lax.fori_loop(0, nv, _hb, 0)

        h = hist[...]
        hp = (h + 15) & jnp.full((16,), ~15, jnp.int32)
        off = plsc.cumsum(hp) - hp
        ptrv[...] = off

        def _sg(st, _):
            cs = jnp.sum(jnp.where(lane == st, h, _i0()))
            os_ = jnp.sum(jnp.where(lane == st, off, _i0()))
            smem[3 + NST + st] = cs
            smem[3 + 2 * NST + st] = os_
            return 0
        lax.fori_loop(0, NST, _sg, 0)

        def _pb(j, _):
            rel = qrel[pl.ds(j * 16, 16)]
            pid = qpid[pl.ds(j * 16, 16)]
            valid = (lane + j * 16) < qn
            _, vs, sstv, svalid, last, run_start, runlen = _runinfo(rel, valid)
            base = plsc.load_gather(ptrv, [sstv], mask=svalid)
            pos = base + (lane - run_start)
            plsc.store_scatter(ptrv, [sstv], base + runlen, mask=last)
            plsc.store_scatter(tmp, [vs + 16], pos, mask=svalid)
            posv = plsc.load_gather(tmp, [lane + 16], mask=valid)
            plsc.store_scatter(srtR, [posv], rel, mask=valid)
            plsc.store_scatter(srtP, [posv], pid, mask=valid)
            return 0
        lax.fori_loop(0, nv, _pb, 0)

        def _stbody(st, _):
            smem[0] = 0
            cnt_st = smem[3 + NST + st]
            off_st = pl.multiple_of(smem[3 + 2 * NST + st], 16)
            nchunks = (cnt_st + CCAP - 1) // CCAP

            def _ck(g, _):
                cbase = off_st + g * CCAP
                cn = jnp.minimum(cnt_st - g * CCAP, CCAP)
                _process_chunk(cbase, cn, st)
                return 0
            lax.fori_loop(0, nchunks, _ck, 0)
            _ensure_slab(st)
            if final:
                def _ob(j, _):
                    zj = lax.shift_right_logical(j, 4)
                    yg = (j & 15) * 16
                    cv = plsc.load_gather(cnt, [(lane + yg) * H + zj])
                    occb[zj, pl.ds(yg, 16)] = jnp.where(
                        cv >= MINPTS, jnp.full((16,), 1.0, jnp.float32),
                        jnp.zeros((16,), jnp.float32))
                    return 0
                lax.fori_loop(0, (H * L) // 16, _ob, 0)
            for src, dst in _out_copies(st, final):
                pltpu.make_async_copy(src, dst, sem).start()
            smem[1] = st
            smem[2] = jnp.int32(1 if final else 0)
            smem[3 + st] = 1
            return 0
        lax.fori_loop(0, NST, _stbody, 0)
        return jnp.int32(0)

    NCH = N // CH_B

    def _scopy(k, h):
        return pltpu.make_async_copy(
            idx_hbm.at[pl.ds(b * N + k * CH_B, CH_B)],
            sbuf.at[pl.ds(h * CH_B, CH_B)], ssem)

    _scopy(0, 0).start()

    def _chunk(jo, qnv):
        for h in range(2):
            k = jo * 2 + h
            _scopy(k, h).wait()

            @pl.when(k + 1 < NCH)
            def _():
                _scopy(k + 1, (h + 1) % 2).start()

            def _fbody(j, qnv):
                for u in range(4):
                    s = pl.ds(h * CH_B + (j * 4 + u) * 16, 16)
                    rel = sbuf[s] - lo
                    m = (rel >= 0) & (rel < VPT)
                    pidv = lane + (b * N + k * CH_B + (j * 4 + u) * 16)
                    mi = _mask_i32(m)
                    inc = plsc.cumsum(mi)
                    pos = qnv + inc - mi
                    plsc.store_scatter(qrel, [pos], rel, mask=m)
                    plsc.store_scatter(qpid, [pos], pidv, mask=m)
                    qnv = qnv + plsc.all_reduce_population_count(m)
                return qnv
            qnv = lax.fori_loop(0, CH_B // 64, _fbody, qnv)
            qs = jnp.max(qnv)

            def _doflush(q):
                _flush(q, False)
                return _i0()
            qnv = lax.cond(qs > QTRIG, _doflush, lambda q: qnv, qs)
        return qnv

    qnv = lax.fori_loop(0, N // CH_B // 2, _chunk, _i0())
    _flush(jnp.max(qnv), True)
    _drain_prev()


@functools.partial(jax.jit, static_argnums=())
def _run(pc1, pa1, prm):
    mesh = plsc.VectorSubcoreMesh(core_axis_name="c", subcore_axis_name="s")
    f = pl.kernel(
        _body,
        mesh=mesh,
        compiler_params=pltpu.CompilerParams(
            needs_layout_passes=False, use_tc_tiling_on_sc=False),
        out_type=[
            jax.ShapeDtypeStruct((NB, C, W, H, L), jnp.float32),
            jax.ShapeDtypeStruct((NB, 1, W, H, L), jnp.float32),
            jax.ShapeDtypeStruct((NB * V,), jnp.int32),
            jax.ShapeDtypeStruct((NB * N,), jnp.int32),
            jax.ShapeDtypeStruct((NB * N, C), jnp.float32),
        ],
        scratch_types=[
            pltpu.VMEM((CH_A,), jnp.float32),
            pltpu.VMEM((CH_A,), jnp.float32),
            pltpu.VMEM((CH_A,), jnp.float32),
            pltpu.VMEM((CH_A,), jnp.int32),
            pltpu.VMEM((C * CH_A,), jnp.float32),
            pltpu.VMEM((2 * CH_B,), jnp.int32),
            pltpu.VMEM((QALLOC,), jnp.int32),
            pltpu.VMEM((QALLOC,), jnp.int32),
            pltpu.VMEM((QALLOC,), jnp.int32),
            pltpu.VMEM((QALLOC,), jnp.int32),
            pltpu.VMEM((C, H, L), jnp.float32),
            pltpu.VMEM((ST,), jnp.int32),
            pltpu.VMEM((H, L), jnp.float32),
            pltpu.VMEM((HASHN,), jnp.int32),
            pltpu.VMEM((16,), jnp.int32),
            pltpu.VMEM((16,), jnp.int32),
            pltpu.VMEM((CCAP, C), jnp.float32),
            pltpu.VMEM((48,), jnp.float32),
            pltpu.VMEM((16,), jnp.float32),
            pltpu.SMEM((3 + 3 * NST,), jnp.int32),
            pltpu.SemaphoreType.DMA,
            pltpu.SemaphoreType.DMA,
            pltpu.SemaphoreType.DMA,
        ],
    )
    return f(pc1, pa1, prm)


def kernel(point_coordinates, point_attributes, origin, voxel_size):
    b, _, rh, rw = point_coordinates.shape
    c = point_attributes.shape[1]
    assert (b, c, rh * rw) == (NB, C, N)
    pc1 = point_coordinates.reshape(b * 3 * N)
    pa1 = point_attributes.reshape(b * c * N)
    prm = jnp.concatenate([
        jnp.broadcast_to(origin.reshape(b * 3, 1), (b * 3, 16)),
        jnp.broadcast_to(voxel_size.reshape(1, 1), (1, 16)),
    ], axis=0).reshape(-1)
    vox, occ, _cnt, _idx, _pat = _run(pc1, pa1, prm)
    return (jnp.transpose(vox, (0, 1, 2, 4, 3)),
            jnp.transpose(occ, (0, 1, 2, 4, 3)))
